# Initial kernel scaffold; baseline (speedup 1.0000x reference)
import jax
import jax.numpy as jnp
from jax.experimental import pallas as pl


def kernel(x, combL_fb, combR_fb, apL_fb, apR_fb, wet1, wet2, dry):
    raise NotImplementedError("write your pallas kernel here")


# speedup vs baseline: 6621.2629x; 6621.2629x over previous
import functools

import jax
import jax.numpy as jnp
from jax import lax
from jax.experimental import pallas as pl
from jax.experimental.pallas import tpu as pltpu
from jax.experimental.pallas import tpu_sc as plsc

_COMB_L = (2205, 2469, 2690, 2998, 3175, 3439, 3627, 4001)
_COMB_R = (2277, 2709, 2924, 3175, 3351, 3487, 3660, 4117)
_AP_L = (556, 441, 341, 225)
_AP_R = (579, 464, 396, 289)
_ALL_D = _COMB_L + _COMB_R + _AP_L + _AP_R

_T = 2048
_B = 16
_NB = _T // _B


def _ceil16(n):
    return -(-n // 16) * 16

_BASE = []
_off = 0
for _d in _ALL_D:
    _BASE.append(_off)
    _off += _ceil16(_d + _T)
_BUF = _off

_AP_Z = _ceil16(max(_AP_L + _AP_R))

_P_WET1, _P_WET2, _P_DRY = 24, 25, 26


def _fv_body(x_hbm, fb_hbm, yl_hbm, yr_hbm, buf, x_v, fb_v, yl_v, yr_v):
    cid = lax.axis_index("c")
    sid = lax.axis_index("s")

    @pl.when(jnp.logical_and(cid == 0, sid == 0))
    def _():
        pltpu.sync_copy(x_hbm, x_v)
        pltpu.sync_copy(fb_hbm, fb_v)

        zero = jnp.zeros((_B,), jnp.float32)
        iota = lax.iota(jnp.int32, _B)

        def zero_combs(i, c):
            off = i * _B
            for f in range(16):
                buf[pl.ds(_BASE[f] + off, _B)] = zero
            return c

        lax.fori_loop(0, _NB, zero_combs, 0)

        def zero_aps(i, c):
            off = i * _B
            for f in range(16, 24):
                buf[pl.ds(_BASE[f] + off, _B)] = zero
            return c

        lax.fori_loop(0, _AP_Z // _B, zero_aps, 0)

        def pget(i):
            return fb_v[pl.ds(i * _B, _B)]

        cfb = [pget(j) for j in range(16)]
        afb = [pget(16 + j) for j in range(8)]
        wet1 = pget(_P_WET1)
        wet2 = pget(_P_WET2)
        dry = pget(_P_DRY)

        def step(i, c):
            t0 = i * _B
            xb = x_v[pl.ds(t0, _B)]
            inp = xb * 0.015
            s_l = zero
            s_r = zero
            for j in range(8):
                b, d = _BASE[j], _ALL_D[j]
                o = buf[pl.ds(b + t0, _B)]
                buf[pl.ds(b + d + t0, _B)] = inp + o * cfb[j]
                s_l = s_l + o
            for j in range(8):
                b, d = _BASE[8 + j], _ALL_D[8 + j]
                o = buf[pl.ds(b + t0, _B)]
                buf[pl.ds(b + d + t0, _B)] = inp + o * cfb[8 + j]
                s_r = s_r + o
            for j in range(4):
                b, d = _BASE[16 + j], _ALL_D[16 + j]
                bo = buf[pl.ds(b + t0, _B)]
                buf[pl.ds(b + d + t0, _B)] = s_l + bo * afb[j]
                s_l = bo - s_l
            for j in range(4):
                b, d = _BASE[20 + j], _ALL_D[20 + j]
                bo = buf[pl.ds(b + t0, _B)]
                buf[pl.ds(b + d + t0, _B)] = s_r + bo * afb[4 + j]
                s_r = bo - s_r
            yl_v[pl.ds(t0, _B)] = s_l * wet1 + s_r * wet2 + xb * dry
            yr_v[pl.ds(t0, _B)] = s_r * wet1 + s_l * wet2 + xb * dry
            return c

        lax.fori_loop(0, _NB, step, 0)

        pltpu.sync_copy(yl_v, yl_hbm)
        pltpu.sync_copy(yr_v, yr_hbm)


_fv_call = functools.partial(
    pl.kernel,
    out_type=[jax.ShapeDtypeStruct((_T,), jnp.float32)] * 2,
    mesh=plsc.VectorSubcoreMesh(
        core_axis_name="c", subcore_axis_name="s", num_cores=2, num_subcores=16
    ),
    compiler_params=pltpu.CompilerParams(needs_layout_passes=False),
    scratch_types=[
        pltpu.VMEM((_BUF,), jnp.float32),
        pltpu.VMEM((_T,), jnp.float32),
        pltpu.VMEM((32 * _B,), jnp.float32),
        pltpu.VMEM((_T,), jnp.float32),
        pltpu.VMEM((_T,), jnp.float32),
    ],
)(_fv_body)


@jax.jit
def kernel(x, combL_fb, combR_fb, apL_fb, apR_fb, wet1, wet2, dry):
    fbs = jnp.concatenate(
        [
            combL_fb.astype(jnp.float32),
            combR_fb.astype(jnp.float32),
            apL_fb.astype(jnp.float32),
            apR_fb.astype(jnp.float32),
            jnp.stack([wet1, wet2, dry]).astype(jnp.float32),
            jnp.zeros((5,), jnp.float32),
        ]
    )
    fbs = jnp.repeat(fbs, _B)
    y_l, y_r = _fv_call(x.astype(jnp.float32), fbs)
    return jnp.stack([y_l, y_r], axis=1)


# speedup vs baseline: 6667.7586x; 1.0070x over previous
import functools

import jax
import jax.numpy as jnp
from jax import lax
from jax.experimental import pallas as pl
from jax.experimental.pallas import tpu as pltpu
from jax.experimental.pallas import tpu_sc as plsc

_COMB_L = (2205, 2469, 2690, 2998, 3175, 3439, 3627, 4001)
_COMB_R = (2277, 2709, 2924, 3175, 3351, 3487, 3660, 4117)
_AP_L = (556, 441, 341, 225)
_AP_R = (579, 464, 396, 289)
_ALL_D = _COMB_L + _COMB_R + _AP_L + _AP_R

_T = 2048
_B = 16
_NB = _T // _B


def _ceil16(n):
    return -(-n // 16) * 16

_AP_Z = _ceil16(max(_AP_L + _AP_R))

_P_WET1, _P_WET2, _P_DRY = 24, 25, 26


def _fv_body(x_hbm, fb_hbm, yl_hbm, yr_hbm, x_v, fb_v, yl_v, yr_v, *rows):
    cid = lax.axis_index("c")
    sid = lax.axis_index("s")

    @pl.when(jnp.logical_and(cid == 0, sid == 0))
    def _():
        pltpu.sync_copy(x_hbm, x_v)
        pltpu.sync_copy(fb_hbm, fb_v)

        zero = jnp.zeros((_B,), jnp.float32)

        def zero_combs(i, c):
            off = i * _B
            for f in range(16):
                rows[f][pl.ds(off, _B)] = zero
            return c

        lax.fori_loop(0, _NB, zero_combs, 0)

        def zero_aps(i, c):
            off = i * _B
            for f in range(16, 24):
                rows[f][pl.ds(off, _B)] = zero
            return c

        lax.fori_loop(0, _AP_Z // _B, zero_aps, 0)

        def pget(i):
            return fb_v[pl.ds(i * _B, _B)]

        cfb = [pget(j) for j in range(16)]
        afb = [pget(16 + j) for j in range(8)]
        wet1 = pget(_P_WET1)
        wet2 = pget(_P_WET2)
        dry = pget(_P_DRY)

        def step(i, c):
            t0 = i * _B
            xb = x_v[pl.ds(t0, _B)]
            inp = xb * 0.015
            s_l = zero
            s_r = zero
            for j in range(8):
                d = _ALL_D[j]
                o = rows[j][pl.ds(t0, _B)]
                rows[j][pl.ds(d + t0, _B)] = inp + o * cfb[j]
                s_l = s_l + o
            for j in range(8):
                d = _ALL_D[8 + j]
                o = rows[8 + j][pl.ds(t0, _B)]
                rows[8 + j][pl.ds(d + t0, _B)] = inp + o * cfb[8 + j]
                s_r = s_r + o
            for j in range(4):
                d = _ALL_D[16 + j]
                bo = rows[16 + j][pl.ds(t0, _B)]
                rows[16 + j][pl.ds(d + t0, _B)] = s_l + bo * afb[j]
                s_l = bo - s_l
            for j in range(4):
                d = _ALL_D[20 + j]
                bo = rows[20 + j][pl.ds(t0, _B)]
                rows[20 + j][pl.ds(d + t0, _B)] = s_r + bo * afb[4 + j]
                s_r = bo - s_r
            yl_v[pl.ds(t0, _B)] = s_l * wet1 + s_r * wet2 + xb * dry
            yr_v[pl.ds(t0, _B)] = s_r * wet1 + s_l * wet2 + xb * dry
            return c

        lax.fori_loop(0, _NB, step, 0)

        pltpu.sync_copy(yl_v, yl_hbm)
        pltpu.sync_copy(yr_v, yr_hbm)


_fv_call = functools.partial(
    pl.kernel,
    out_type=[jax.ShapeDtypeStruct((_T,), jnp.float32)] * 2,
    mesh=plsc.VectorSubcoreMesh(
        core_axis_name="c", subcore_axis_name="s", num_cores=2, num_subcores=16
    ),
    compiler_params=pltpu.CompilerParams(needs_layout_passes=False),
    scratch_types=[
        pltpu.VMEM((_T,), jnp.float32),
        pltpu.VMEM((32 * _B,), jnp.float32),
        pltpu.VMEM((_T,), jnp.float32),
        pltpu.VMEM((_T,), jnp.float32),
    ]
    + [pltpu.VMEM((_ceil16(_d + _T),), jnp.float32) for _d in _ALL_D],
)(_fv_body)


@jax.jit
def kernel(x, combL_fb, combR_fb, apL_fb, apR_fb, wet1, wet2, dry):
    fbs = jnp.concatenate(
        [
            combL_fb.astype(jnp.float32),
            combR_fb.astype(jnp.float32),
            apL_fb.astype(jnp.float32),
            apR_fb.astype(jnp.float32),
            jnp.stack([wet1, wet2, dry]).astype(jnp.float32),
            jnp.zeros((5,), jnp.float32),
        ]
    )
    fbs = jnp.repeat(fbs, _B)
    y_l, y_r = _fv_call(x.astype(jnp.float32), fbs)
    return jnp.stack([y_l, y_r], axis=1)
